# Initial kernel scaffold; baseline (speedup 1.0000x reference)
import jax
import jax.numpy as jnp
from jax.experimental import pallas as pl


def kernel(coord, img_feat_0, img_feat_1, img_feat_2, img_feat_3):
    raise NotImplementedError("write your pallas kernel here")


# speedup vs baseline: 3.4302x; 3.4302x over previous
import functools

import jax
import jax.numpy as jnp
from jax import lax
from jax.experimental import pallas as pl
from jax.experimental.pallas import tpu as pltpu
from jax.experimental.pallas import tpu_sc as plsc

N_POINTS = 100000
CHUNK = 16
N_CHUNKS = N_POINTS // CHUNK
N_WORKERS = 32
CHUNKS_PER_WORKER = -(-N_CHUNKS // N_WORKERS)

SCALES = ((56, 64, 3), (28, 128, 67), (14, 256, 195), (7, 512, 451))
OUT_COLS = 963


def _tec_kernel(x_hbm, y_hbm, z_hbm, f0_hbm, f1_hbm, f2_hbm, f3_hbm, out_hbm,
                xb, yb, zb, wbuf, idx0, idx1, idx2, idx3, q0, q1, q2, q3,
                outbuf, sem0, sem1, sem2, sem3):
    wid = lax.axis_index("c") * 16 + lax.axis_index("s")
    iota = lax.iota(jnp.int32, CHUNK)
    feats = (f0_hbm, f1_hbm, f2_hbm, f3_hbm)
    idxs = (idx0, idx1, idx2, idx3)
    qs = (q0, q1, q2, q3)
    sems = (sem0, sem1, sem2, sem3)

    def chunk_body(k, carry):
        chunk = k * N_WORKERS + wid

        @pl.when(chunk < N_CHUNKS)
        def _():
            base = chunk * CHUNK
            pltpu.sync_copy(x_hbm.at[pl.ds(base, CHUNK)], xb)
            pltpu.sync_copy(y_hbm.at[pl.ds(base, CHUNK)], yb)
            pltpu.sync_copy(z_hbm.at[pl.ds(base, CHUNK)], zb)
            xv = xb[...]
            yv = yb[...]
            zv = zb[...]

            h = 112.0 * ((-yv) / (-zv)) + 111.5
            w = 112.0 * (xv / (-zv)) + 111.5
            h = jnp.minimum(jnp.maximum(h, 0.0), 223.0)
            w = jnp.minimum(jnp.maximum(w, 0.0), 223.0)

            for s, (g, c, _off) in enumerate(SCALES):
                sx = h * (g / 224.0)
                sy = w * (g / 224.0)
                i1 = sx.astype(jnp.int32)
                j1 = sy.astype(jnp.int32)
                fx = sx - i1.astype(jnp.float32)
                fy = sy - j1.astype(jnp.float32)
                wx_hi = fx
                wx_lo = jnp.where(fx > 0.0, 1.0 - fx, 0.0)
                wy_hi = fy
                wy_lo = jnp.where(fy > 0.0, 1.0 - fy, 0.0)
                i2 = jnp.minimum(i1 + 1, g - 1)
                j2 = jnp.minimum(j1 + 1, g - 1)
                ib = idxs[s]
                ib[pl.ds(0, CHUNK)] = i1 * g + j1
                ib[pl.ds(16, CHUNK)] = i2 * g + j1
                ib[pl.ds(32, CHUNK)] = i1 * g + j2
                ib[pl.ds(48, CHUNK)] = i2 * g + j2
                wbuf[pl.ds(s * 64 + 0, CHUNK)] = wx_lo * wy_lo
                wbuf[pl.ds(s * 64 + 16, CHUNK)] = wx_hi * wy_lo
                wbuf[pl.ds(s * 64 + 32, CHUNK)] = wx_lo * wy_hi
                wbuf[pl.ds(s * 64 + 48, CHUNK)] = wx_hi * wy_hi

            handles = [
                pltpu.async_copy(feats[s].at[idxs[s]], qs[s], sems[s])
                for s in range(4)
            ]
            for hdl in handles:
                hdl.wait()

            plsc.store_scatter(outbuf, [iota * OUT_COLS], xv)
            plsc.store_scatter(outbuf, [iota * OUT_COLS + 1], yv)
            plsc.store_scatter(outbuf, [iota * OUT_COLS + 2], zv)

            zeros = jnp.zeros((CHUNK,), jnp.int32)

            def point_body(p, carry2):
                row = p * OUT_COLS
                for s, (g, c, off) in enumerate(SCALES):
                    q = qs[s]
                    wp = zeros + (s * 64 + p)
                    w11v = plsc.load_gather(wbuf, [wp])
                    w21v = plsc.load_gather(wbuf, [wp + 16])
                    w12v = plsc.load_gather(wbuf, [wp + 32])
                    w22v = plsc.load_gather(wbuf, [wp + 48])
                    for c0 in range(0, c, CHUNK):
                        v0 = q[p, pl.ds(c0, CHUNK)]
                        v1 = q[16 + p, pl.ds(c0, CHUNK)]
                        v2 = q[32 + p, pl.ds(c0, CHUNK)]
                        v3 = q[48 + p, pl.ds(c0, CHUNK)]
                        acc = w11v * v0 + w21v * v1 + w12v * v2 + w22v * v3
                        plsc.store_scatter(
                            outbuf, [iota + (row + off + c0)], acc)
                return carry2

            lax.fori_loop(0, CHUNK, point_body, 0)

            pltpu.sync_copy(outbuf,
                            out_hbm.at[pl.ds(base * OUT_COLS,
                                             CHUNK * OUT_COLS)])

        return carry

    lax.fori_loop(0, CHUNKS_PER_WORKER, chunk_body, 0)


@jax.jit
def kernel(coord, img_feat_0, img_feat_1, img_feat_2, img_feat_3):
    x = coord[:, 0]
    y = coord[:, 1]
    z = coord[:, 2]
    f0 = img_feat_0.reshape(56 * 56, 64)
    f1 = img_feat_1.reshape(28 * 28, 128)
    f2 = img_feat_2.reshape(14 * 14, 256)
    f3 = img_feat_3.reshape(7 * 7, 512)

    run = functools.partial(
        pl.kernel,
        mesh=plsc.VectorSubcoreMesh(core_axis_name="c", subcore_axis_name="s"),
        compiler_params=pltpu.CompilerParams(needs_layout_passes=False,
                                             use_tc_tiling_on_sc=False),
        out_type=jax.ShapeDtypeStruct((N_POINTS * OUT_COLS,), jnp.float32),
        scratch_types=[
            pltpu.VMEM((CHUNK,), jnp.float32),
            pltpu.VMEM((CHUNK,), jnp.float32),
            pltpu.VMEM((CHUNK,), jnp.float32),
            pltpu.VMEM((256,), jnp.float32),
            pltpu.VMEM((64,), jnp.int32),
            pltpu.VMEM((64,), jnp.int32),
            pltpu.VMEM((64,), jnp.int32),
            pltpu.VMEM((64,), jnp.int32),
            pltpu.VMEM((64, 64), jnp.float32),
            pltpu.VMEM((64, 128), jnp.float32),
            pltpu.VMEM((64, 256), jnp.float32),
            pltpu.VMEM((64, 512), jnp.float32),
            pltpu.VMEM((CHUNK * OUT_COLS,), jnp.float32),
            pltpu.SemaphoreType.DMA,
            pltpu.SemaphoreType.DMA,
            pltpu.SemaphoreType.DMA,
            pltpu.SemaphoreType.DMA,
        ],
    )(_tec_kernel)
    flat = run(x, y, z, f0, f1, f2, f3)
    return flat.reshape(N_POINTS, OUT_COLS)
